# baseline (speedup 1.0000x reference)
import functools

import jax
import jax.numpy as jnp
from jax import lax
from jax.experimental import pallas as pl
from jax.experimental.pallas import tpu as pltpu
from jax.experimental.pallas import tpu_sc as plsc

NC = 2
NS = 16
NW = NC * NS
L = 16
M = 65536
D = 128
RPW = M // NW
IMAX = 2147483647

BLK = 8192
NBLK = M // BLK
UCB = 8192
NUCB = M // UCB

_mesh = plsc.VectorSubcoreMesh(core_axis_name="c", subcore_axis_name="s")
_params = pltpu.CompilerParams(needs_layout_passes=False,
                               skip_device_barrier=True)


def _uccopy_body(uc_hbm, ucout_hbm, ucb, semu):
    cid = lax.axis_index("c")
    sid = lax.axis_index("s")
    wid = sid * NC + cid
    wbase = wid * RPW
    pltpu.async_copy(uc_hbm.at[pl.ds(wbase, RPW)], ucb, semu).wait()
    pltpu.sync_copy(ucb, ucout_hbm.at[pl.ds(wbase, RPW)])


_uccopy = functools.partial(
    pl.kernel,
    out_type=jax.ShapeDtypeStruct((M,), jnp.int32),
    mesh=_mesh,
    compiler_params=_params,
    scratch_types=[
        pltpu.VMEM((RPW,), jnp.int32),
        pltpu.SemaphoreType.DMA,
    ],
)(_uccopy_body)


def _mv_body(q_ref, k_ref, s_ref):
    s_ref[...] = jax.lax.dot_general(
        q_ref[...], k_ref[...],
        dimension_numbers=(((1,), (1,)), ((), ())),
        preferred_element_type=jnp.float32)


_mv = pl.pallas_call(
    _mv_body,
    grid=(NBLK,),
    in_specs=[
        pl.BlockSpec((1, D), lambda b: (0, 0)),
        pl.BlockSpec((BLK, D), lambda b: (b, 0)),
    ],
    out_specs=pl.BlockSpec((1, BLK), lambda b: (0, b)),
    out_shape=jax.ShapeDtypeStruct((1, M), jnp.float32),
)


def _amax_body(s_ref, a_ref, p_ref, bm_ref, bi_ref):
    s = jnp.where(a_ref[...], s_ref[...], -jnp.inf)
    m = jnp.max(s)
    io = lax.broadcasted_iota(jnp.int32, (1, M), 1)
    idx = jnp.min(jnp.where(s == m, io, IMAX))
    npc = jnp.where(m > 0.95, idx, p_ref[0, 0])
    io1 = lax.broadcasted_iota(jnp.int32, (1, D), 1)
    bm_ref[...] = jnp.broadcast_to(m, (1, D))
    bi_ref[...] = jnp.where(io1 == 0, idx, jnp.where(io1 == 1, npc, 0))


_amax = pl.pallas_call(
    _amax_body,
    out_shape=[
        jax.ShapeDtypeStruct((1, D), jnp.float32),
        jax.ShapeDtypeStruct((1, D), jnp.int32),
    ],
)


def _touch_body(s_ref, uc_ref, av_ref, ucout_ref, act_ref):
    idx = s_ref[0]
    hit = s_ref[1]
    b = pl.program_id(0)
    io = lax.broadcasted_iota(jnp.int32, (1, 1, UCB), 2) + b * UCB
    ucout_ref[...] = uc_ref[...] + (
        (io == idx) & (hit > 0)).astype(jnp.int32)
    io0 = lax.broadcasted_iota(jnp.int32, (8, D), 0)
    act_ref[...] = jnp.sum(
        jnp.where(io0 == idx % 8, av_ref[...], 0.0), axis=0, keepdims=True)


_touch = pl.pallas_call(
    _touch_body,
    grid_spec=pltpu.PrefetchScalarGridSpec(
        num_scalar_prefetch=1,
        grid=(NUCB,),
        in_specs=[
            pl.BlockSpec((1, 1, UCB), lambda b, s: (b, 0, 0)),
            pl.BlockSpec((8, D), lambda b, s: (s[0] // 8, 0)),
        ],
        out_specs=[
            pl.BlockSpec((1, 1, UCB), lambda b, s: (b, 0, 0)),
            pl.BlockSpec((1, D), lambda b, s: (0, 0)),
        ],
    ),
    out_shape=[
        jax.ShapeDtypeStruct((NUCB, 1, UCB), jnp.int32),
        jax.ShapeDtypeStruct((1, D), jnp.float32),
    ],
)


def kernel(sensor_spikes, sensor_keys, action_values, is_active,
           usage_counts, program_counter):
    pc11 = program_counter.reshape(1, 1).astype(jnp.int32)

    uc0 = _uccopy(usage_counts)
    sim = _mv(sensor_spikes, sensor_keys)
    bm, bi = _amax(sim, is_active.reshape(1, M), pc11)

    sarg = jnp.concatenate(
        [bi[0, 0:1], (bm[0, 0:1] > 0.95).astype(jnp.int32)])
    ucn3, act = _touch(sarg, uc0.reshape(NUCB, 1, UCB), action_values)

    action = act.reshape(1, D)
    confidence = bm[0, 0:1]
    best_idx = bi[0, 0:1]
    new_pc = bi[0, 1]
    return action, confidence, best_idx, ucn3.reshape(M), new_pc


# speedup vs baseline: 1.0965x; 1.0965x over previous
import functools

import jax
import jax.numpy as jnp
from jax import lax
from jax.experimental import pallas as pl
from jax.experimental.pallas import tpu as pltpu
from jax.experimental.pallas import tpu_sc as plsc

NC = 2
NS = 16
NW = NC * NS
L = 16
M = 65536
D = 128
RPW = M // NW
IMAX = 2147483647

BLK = 8192
NBLK = M // BLK
UCB = 32768
NUCB = M // UCB

_mesh = plsc.VectorSubcoreMesh(core_axis_name="c", subcore_axis_name="s")
_params = pltpu.CompilerParams(needs_layout_passes=False,
                               skip_device_barrier=True)


def _uccopy_body(uc_hbm, ucout_hbm, ucb, semu):
    cid = lax.axis_index("c")
    sid = lax.axis_index("s")
    wid = sid * NC + cid
    wbase = wid * RPW
    pltpu.async_copy(uc_hbm.at[pl.ds(wbase, RPW)], ucb, semu).wait()
    pltpu.sync_copy(ucb, ucout_hbm.at[pl.ds(wbase, RPW)])


_uccopy = functools.partial(
    pl.kernel,
    out_type=jax.ShapeDtypeStruct((M,), jnp.int32),
    mesh=_mesh,
    compiler_params=_params,
    scratch_types=[
        pltpu.VMEM((RPW,), jnp.int32),
        pltpu.SemaphoreType.DMA,
    ],
)(_uccopy_body)


def _mv_body(q_ref, k_ref, s_ref):
    s_ref[...] = jax.lax.dot_general(
        q_ref[...], k_ref[...],
        dimension_numbers=(((1,), (1,)), ((), ())),
        preferred_element_type=jnp.float32)


_mv = pl.pallas_call(
    _mv_body,
    grid=(NBLK,),
    in_specs=[
        pl.BlockSpec((1, D), lambda b: (0, 0)),
        pl.BlockSpec((BLK, D), lambda b: (b, 0)),
    ],
    out_specs=pl.BlockSpec((1, BLK), lambda b: (0, b)),
    out_shape=jax.ShapeDtypeStruct((1, M), jnp.float32),
)


def _amax_body(s_ref, a_ref, p_ref, bm_ref, bi_ref):
    s = jnp.where(a_ref[...], s_ref[...], -jnp.inf)
    m = jnp.max(s)
    io = lax.broadcasted_iota(jnp.int32, (1, M), 1)
    idx = jnp.min(jnp.where(s == m, io, IMAX))
    hit = (m > 0.95).astype(jnp.int32)
    npc = jnp.where(m > 0.95, idx, p_ref[0, 0])
    io1 = lax.broadcasted_iota(jnp.int32, (1, D), 1)
    bm_ref[...] = jnp.broadcast_to(m, (1, D))
    bi_ref[...] = jnp.where(
        io1 == 0, idx, jnp.where(io1 == 1, hit, jnp.where(io1 == 2, npc, 0)))


_amax = pl.pallas_call(
    _amax_body,
    out_shape=[
        jax.ShapeDtypeStruct((1, D), jnp.float32),
        jax.ShapeDtypeStruct((1, D), jnp.int32),
    ],
)


def _touch_body(s_ref, uc_ref, av_ref, ucout_ref, act_ref):
    idx = s_ref[0]
    hit = s_ref[1]
    b = pl.program_id(0)
    io = lax.broadcasted_iota(jnp.int32, (1, 1, UCB), 2) + b * UCB
    ucout_ref[...] = uc_ref[...] + (
        (io == idx) & (hit > 0)).astype(jnp.int32)
    io0 = lax.broadcasted_iota(jnp.int32, (8, D), 0)
    act_ref[...] = jnp.sum(
        jnp.where(io0 == idx % 8, av_ref[...], 0.0), axis=0, keepdims=True)


_touch = pl.pallas_call(
    _touch_body,
    grid_spec=pltpu.PrefetchScalarGridSpec(
        num_scalar_prefetch=1,
        grid=(NUCB,),
        in_specs=[
            pl.BlockSpec((1, 1, UCB), lambda b, s: (b, 0, 0)),
            pl.BlockSpec((8, D), lambda b, s: (s[0] // 8, 0)),
        ],
        out_specs=[
            pl.BlockSpec((1, 1, UCB), lambda b, s: (b, 0, 0)),
            pl.BlockSpec((1, D), lambda b, s: (0, 0)),
        ],
    ),
    out_shape=[
        jax.ShapeDtypeStruct((NUCB, 1, UCB), jnp.int32),
        jax.ShapeDtypeStruct((1, D), jnp.float32),
    ],
)


def kernel(sensor_spikes, sensor_keys, action_values, is_active,
           usage_counts, program_counter):
    pc11 = program_counter.reshape(1, 1).astype(jnp.int32)

    uc0 = _uccopy(usage_counts)
    sim = _mv(sensor_spikes, sensor_keys)
    bm, bi = _amax(sim, is_active.reshape(1, M), pc11)

    sarg = bi[0, 0:2]
    ucn3, act = _touch(sarg, uc0.reshape(NUCB, 1, UCB), action_values)

    action = act.reshape(1, D)
    confidence = bm[0, 0:1]
    best_idx = bi[0, 0:1]
    new_pc = bi[0, 2]
    return action, confidence, best_idx, ucn3.reshape(M), new_pc
